# baseline (speedup 1.0000x reference)
import functools

import jax
import jax.numpy as jnp
from jax import lax
from jax.experimental import pallas as pl
from jax.experimental.pallas import tpu as pltpu
from jax.experimental.pallas import tpu_sc as plsc

N_BINS = 8
OUT = 224
H = W = 512
P = OUT * OUT


def _interp_matrix(lo, frac, size):
    cols = lax.broadcasted_iota(jnp.int32, (OUT, size), 1)
    hi = jnp.minimum(lo + 1, size - 1)
    return (jnp.where(cols == lo, 1.0 - frac, 0.0)
            + jnp.where(cols == hi, frac, 0.0))


def _coords(start, extent, size):
    g = (lax.broadcasted_iota(jnp.int32, (OUT, 1), 0).astype(jnp.float32)
         + 0.5) / OUT
    c = jnp.clip(start + g * extent, 0.0, size - 1.0)
    c0 = jnp.floor(c)
    return c0.astype(jnp.int32), c - c0


def _tc_body(boxes_ref, flows_ref, mag_ref, bins_ref):
    m = pl.program_id(0)
    bidx = boxes_ref[m, 0].astype(jnp.int32)
    x1 = boxes_ref[m, 1]
    y1 = boxes_ref[m, 2]
    roi_w = jnp.maximum(boxes_ref[m, 3] - x1, 1.0)
    roi_h = jnp.maximum(boxes_ref[m, 4] - y1, 1.0)

    y0i, ly = _coords(y1, roi_h, H)
    x0i, lx = _coords(x1, roi_w, W)
    wy = _interp_matrix(y0i, ly, H)
    wx = _interp_matrix(x0i, lx, W)

    def sample(c):
        img = flows_ref[bidx, c]
        tmp = lax.dot_general(wy, img, (((1,), (0,)), ((), ())),
                              preferred_element_type=jnp.float32)
        return lax.dot_general(tmp, wx, (((1,), (1,)), ((), ())),
                               preferred_element_type=jnp.float32)

    a = sample(0)
    b = sample(1)
    mag_ref[0] = jnp.sqrt(a * a + b * b)
    theta = jnp.arctan2(a, b)
    bins_ref[0] = jnp.clip(
        jnp.floor((theta + jnp.pi) / (2.0 * jnp.pi) * N_BINS),
        0, N_BINS - 1).astype(jnp.int32)


_info = plsc.get_sparse_core_info()
_NC, _NS = _info.num_cores, _info.num_subcores
_NW = _NC * _NS


def _sc_body(mag_hbm, bins_hbm, out_hbm, magv, binv, histf, cntf, resv):
    wid = lax.axis_index("s") * _NC + lax.axis_index("c")
    lanes = jnp.arange(16, dtype=jnp.int32)
    ones = jnp.ones((16,), jnp.float32)
    zeros = jnp.zeros((16,), jnp.float32)
    M = mag_hbm.shape[0]
    per_w = M // _NW

    for k in range(per_w):
        box = wid * per_w + k
        pltpu.sync_copy(mag_hbm.at[box], magv)
        pltpu.sync_copy(bins_hbm.at[box], binv)
        for i in range(8):
            histf[pl.ds(i * 16, 16)] = zeros
            cntf[pl.ds(i * 16, 16)] = zeros

        def body(i, carry):
            off = i * 16
            mg = magv[pl.ds(off, 16)]
            bn = binv[pl.ds(off, 16)]
            flat = lanes * N_BINS + bn
            plsc.addupdate_scatter(histf, [flat], mg)
            plsc.addupdate_scatter(cntf, [flat], ones)
            return carry

        lax.fori_loop(0, P // 16, body, 0)

        hacc = zeros
        cacc = zeros
        for i in range(8):
            hacc = hacc + histf[pl.ds(i * 16, 16)]
            cacc = cacc + cntf[pl.ds(i * 16, 16)]
        histf[pl.ds(0, 16)] = hacc
        cntf[pl.ds(0, 16)] = cacc
        rot = (lanes + 8) % 16
        htot = hacc + plsc.load_gather(histf, [rot])
        ctot = cacc + plsc.load_gather(cntf, [rot])
        nz = ctot != 0.0
        res = jnp.where(nz, htot / jnp.where(nz, ctot, 1.0), 0.0)
        resv[...] = res
        pltpu.sync_copy(resv, out_hbm.at[box])


def _sc_hist(mag, bins):
    M = mag.shape[0]
    return pl.kernel(
        _sc_body,
        mesh=plsc.VectorSubcoreMesh(core_axis_name="c", subcore_axis_name="s"),
        out_type=jax.ShapeDtypeStruct((M, 16), jnp.float32),
        scratch_types=[
            pltpu.VMEM((P,), jnp.float32),
            pltpu.VMEM((P,), jnp.int32),
            pltpu.VMEM((16 * N_BINS,), jnp.float32),
            pltpu.VMEM((16 * N_BINS,), jnp.float32),
            pltpu.VMEM((16,), jnp.float32),
        ],
        compiler_params=pltpu.CompilerParams(needs_layout_passes=False),
    )(mag, bins)


def kernel(flows, boxes):
    M = boxes.shape[0]
    mag, bins = pl.pallas_call(
        _tc_body,
        grid=(M,),
        in_specs=[
            pl.BlockSpec(memory_space=pltpu.SMEM),
            pl.BlockSpec((flows.shape[0], 2, H, W), lambda m: (0, 0, 0, 0)),
        ],
        out_specs=[
            pl.BlockSpec((1, OUT, OUT), lambda m: (m, 0, 0)),
            pl.BlockSpec((1, OUT, OUT), lambda m: (m, 0, 0)),
        ],
        out_shape=[
            jax.ShapeDtypeStruct((M, OUT, OUT), jnp.float32),
            jax.ShapeDtypeStruct((M, OUT, OUT), jnp.int32),
        ],
    )(boxes, flows)
    out = _sc_hist(mag.reshape(M, P), bins.reshape(M, P))
    return out[:, :N_BINS]


# speedup vs baseline: 1.0869x; 1.0869x over previous
import jax
import jax.numpy as jnp
from jax import lax
from jax.experimental import pallas as pl
from jax.experimental.pallas import tpu as pltpu
from jax.experimental.pallas import tpu_sc as plsc

N_BINS = 8
OUT = 224
H = W = 512
P = OUT * OUT
CH = 6272
NCHUNK = P // CH
UNROLL = 4


def _interp_matrix(lo, frac, size):
    cols = lax.broadcasted_iota(jnp.int32, (OUT, size), 1)
    hi = jnp.minimum(lo + 1, size - 1)
    return (jnp.where(cols == lo, 1.0 - frac, 0.0)
            + jnp.where(cols == hi, frac, 0.0))


def _coords(start, extent, size):
    g = (lax.broadcasted_iota(jnp.int32, (OUT, 1), 0).astype(jnp.float32)
         + 0.5) / OUT
    c = jnp.clip(start + g * extent, 0.0, size - 1.0)
    c0 = jnp.floor(c)
    return c0.astype(jnp.int32), c - c0


def _octant(a, b):
    sa = a < 0
    sb = (b < 0) | ((b == 0) & (a > 0))
    aa, ab = jnp.abs(a), jnp.abs(b)
    d = (aa > ab) | ((aa == ab) & ~((~sa) & sb))
    t = jnp.where(sb, 2.0, 0.0) + jnp.where(sb != d, 1.0, 0.0)
    return jnp.where(sa, 3.0 - t, 4.0 + t).astype(jnp.int32)


def _tc_body(boxes_ref, flows_ref, mag_ref, bins_ref):
    m = pl.program_id(0)
    bidx = boxes_ref[m, 0].astype(jnp.int32)
    x1 = boxes_ref[m, 1]
    y1 = boxes_ref[m, 2]
    roi_w = jnp.maximum(boxes_ref[m, 3] - x1, 1.0)
    roi_h = jnp.maximum(boxes_ref[m, 4] - y1, 1.0)

    y0i, ly = _coords(y1, roi_h, H)
    x0i, lx = _coords(x1, roi_w, W)
    wy = _interp_matrix(y0i, ly, H).astype(jnp.bfloat16)
    wx = _interp_matrix(x0i, lx, W).astype(jnp.bfloat16)

    def sample(c):
        img = flows_ref[bidx, c]
        tmp = lax.dot_general(wy, img, (((1,), (0,)), ((), ())),
                              preferred_element_type=jnp.float32)
        return lax.dot_general(tmp.astype(jnp.bfloat16), wx,
                               (((1,), (1,)), ((), ())),
                               preferred_element_type=jnp.float32)

    a = sample(0)
    b = sample(1)
    mag_ref[0] = jnp.sqrt(a * a + b * b)
    bins_ref[0] = _octant(a, b)


_info = plsc.get_sparse_core_info()
_NC, _NS = _info.num_cores, _info.num_subcores
_NW = _NC * _NS


def _sc_body(mag_hbm, bins_hbm, out_hbm, magv0, magv1, binv0, binv1,
             histf, cntf, resv, sem0, sem1):
    wid = lax.axis_index("s") * _NC + lax.axis_index("c")
    lanes = jnp.arange(16, dtype=jnp.int32)
    ones = jnp.ones((16,), jnp.float32)
    zeros = jnp.zeros((16,), jnp.float32)
    lane_bases = [lanes * N_BINS + (u % 2) * 16 * N_BINS for u in range(UNROLL)]
    M = mag_hbm.shape[0]
    per_w = M // _NW
    total = per_w * NCHUNK
    sems = (sem0, sem1)
    mags = (magv0, magv1)
    bins_ = (binv0, binv1)
    pend = [[None, None], [None, None]]

    def start(t, buf):
        box = wid * per_w + (t // NCHUNK)
        off = (t % NCHUNK) * CH
        pend[buf][0] = pltpu.async_copy(
            mag_hbm.at[box, pl.ds(off, CH)], mags[buf], sems[buf])
        pend[buf][1] = pltpu.async_copy(
            bins_hbm.at[box, pl.ds(off, CH)], bins_[buf], sems[buf])

    def clear_tables():
        for i in range(16):
            histf[pl.ds(i * 16, 16)] = zeros
            cntf[pl.ds(i * 16, 16)] = zeros

    def fold(tab):
        acc = zeros
        for i in range(16):
            acc = acc + tab[pl.ds(i * 16, 16)]
        tab[pl.ds(0, 16)] = acc
        return acc + plsc.load_gather(tab, [(lanes + 8) % 16])

    clear_tables()
    start(0, 0)
    for t in range(total):
        buf = t % 2
        if t + 1 < total:
            start(t + 1, 1 - buf)
        pend[buf][0].wait()
        pend[buf][1].wait()
        mv, bv = mags[buf], bins_[buf]

        def body(i, carry, mv=mv, bv=bv):
            off = i * (16 * UNROLL)
            for u in range(UNROLL):
                mg = mv[pl.ds(off + u * 16, 16)]
                bn = bv[pl.ds(off + u * 16, 16)]
                flat = lane_bases[u] + bn
                plsc.addupdate_scatter(histf, [flat], mg)
                plsc.addupdate_scatter(cntf, [flat], ones)
            return carry

        lax.fori_loop(0, CH // (16 * UNROLL), body, 0)

        if (t + 1) % NCHUNK == 0:
            box = wid * per_w + (t // NCHUNK)
            htot = fold(histf)
            ctot = fold(cntf)
            nz = ctot != 0.0
            resv[...] = jnp.where(nz, htot / jnp.where(nz, ctot, 1.0), 0.0)
            pltpu.sync_copy(resv, out_hbm.at[box])
            if t + 1 < total:
                clear_tables()


def _sc_hist(mag, bins):
    M = mag.shape[0]
    return pl.kernel(
        _sc_body,
        mesh=plsc.VectorSubcoreMesh(core_axis_name="c", subcore_axis_name="s"),
        out_type=jax.ShapeDtypeStruct((M, 16), jnp.float32),
        scratch_types=[
            pltpu.VMEM((CH,), jnp.float32),
            pltpu.VMEM((CH,), jnp.float32),
            pltpu.VMEM((CH,), jnp.int32),
            pltpu.VMEM((CH,), jnp.int32),
            pltpu.VMEM((2 * 16 * N_BINS,), jnp.float32),
            pltpu.VMEM((2 * 16 * N_BINS,), jnp.float32),
            pltpu.VMEM((16,), jnp.float32),
            pltpu.SemaphoreType.DMA,
            pltpu.SemaphoreType.DMA,
        ],
        compiler_params=pltpu.CompilerParams(needs_layout_passes=False),
    )(mag, bins)


def kernel(flows, boxes):
    M = boxes.shape[0]
    mag, bins = pl.pallas_call(
        _tc_body,
        grid=(M,),
        in_specs=[
            pl.BlockSpec(memory_space=pltpu.SMEM),
            pl.BlockSpec((flows.shape[0], 2, H, W), lambda m: (0, 0, 0, 0)),
        ],
        out_specs=[
            pl.BlockSpec((1, OUT, OUT), lambda m: (m, 0, 0)),
            pl.BlockSpec((1, OUT, OUT), lambda m: (m, 0, 0)),
        ],
        out_shape=[
            jax.ShapeDtypeStruct((M, OUT, OUT), jnp.float32),
            jax.ShapeDtypeStruct((M, OUT, OUT), jnp.int32),
        ],
    )(boxes, flows.astype(jnp.bfloat16))
    out = _sc_hist(mag.reshape(M, P), bins.reshape(M, P))
    return out[:, :N_BINS]


# speedup vs baseline: 1.1513x; 1.0592x over previous
import jax
import jax.numpy as jnp
from jax import lax
from jax.experimental import pallas as pl
from jax.experimental.pallas import tpu as pltpu
from jax.experimental.pallas import tpu_sc as plsc

N_BINS = 8
OUT = 224
H = W = 512
P = OUT * OUT
CH = 25088
NCHUNK = P // CH
UNROLL = 8


def _interp_matrix(lo, frac, size):
    cols = lax.broadcasted_iota(jnp.int32, (OUT, size), 1)
    hi = jnp.minimum(lo + 1, size - 1)
    return (jnp.where(cols == lo, 1.0 - frac, 0.0)
            + jnp.where(cols == hi, frac, 0.0))


def _coords(start, extent, size):
    g = (lax.broadcasted_iota(jnp.int32, (OUT, 1), 0).astype(jnp.float32)
         + 0.5) / OUT
    c = jnp.clip(start + g * extent, 0.0, size - 1.0)
    c0 = jnp.floor(c)
    return c0.astype(jnp.int32), c - c0


def _octant(a, b):
    sa = a < 0
    sb = (b < 0) | ((b == 0) & (a > 0))
    aa, ab = jnp.abs(a), jnp.abs(b)
    d = (aa > ab) | ((aa == ab) & ~((~sa) & sb))
    t = jnp.where(sb, 2.0, 0.0) + jnp.where(sb != d, 1.0, 0.0)
    return jnp.where(sa, 3.0 - t, 4.0 + t).astype(jnp.int32)


def _tc_body(boxes_ref, flows_ref, mag_ref, bins_ref):
    m = pl.program_id(0)
    bidx = boxes_ref[m, 0].astype(jnp.int32)
    x1 = boxes_ref[m, 1]
    y1 = boxes_ref[m, 2]
    roi_w = jnp.maximum(boxes_ref[m, 3] - x1, 1.0)
    roi_h = jnp.maximum(boxes_ref[m, 4] - y1, 1.0)

    y0i, ly = _coords(y1, roi_h, H)
    x0i, lx = _coords(x1, roi_w, W)
    wy = _interp_matrix(y0i, ly, H).astype(jnp.bfloat16)
    wx = _interp_matrix(x0i, lx, W).astype(jnp.bfloat16)

    t0 = lax.dot_general(wy, flows_ref[bidx, 0], (((1,), (0,)), ((), ())),
                         preferred_element_type=jnp.float32)
    t1 = lax.dot_general(wy, flows_ref[bidx, 1], (((1,), (0,)), ((), ())),
                         preferred_element_type=jnp.float32)
    tcat = jnp.concatenate(
        [t0.astype(jnp.bfloat16), t1.astype(jnp.bfloat16)], axis=0)
    r = lax.dot_general(tcat, wx, (((1,), (1,)), ((), ())),
                        preferred_element_type=jnp.float32)
    a = r[:OUT]
    b = r[OUT:]
    mag_ref[0] = jnp.sqrt(a * a + b * b)
    bins_ref[0] = _octant(a, b)


_info = plsc.get_sparse_core_info()
_NC, _NS = _info.num_cores, _info.num_subcores
_NW = _NC * _NS


def _sc_body(mag_hbm, bins_hbm, out_hbm, magv0, magv1, binv0, binv1,
             histf, cntf, resv, sem0, sem1):
    wid = lax.axis_index("s") * _NC + lax.axis_index("c")
    lanes = jnp.arange(16, dtype=jnp.int32)
    ones = jnp.ones((16,), jnp.float32)
    zeros = jnp.zeros((16,), jnp.float32)
    lane_bases = [lanes * N_BINS + (u % 2) * 16 * N_BINS for u in range(UNROLL)]
    M = mag_hbm.shape[0]
    per_w = M // _NW
    total = per_w * NCHUNK
    sems = (sem0, sem1)
    mags = (magv0, magv1)
    bins_ = (binv0, binv1)
    pend = [[None, None], [None, None]]

    def start(t, buf):
        box = wid * per_w + (t // NCHUNK)
        off = (t % NCHUNK) * CH
        pend[buf][0] = pltpu.async_copy(
            mag_hbm.at[box, pl.ds(off, CH)], mags[buf], sems[buf])
        pend[buf][1] = pltpu.async_copy(
            bins_hbm.at[box, pl.ds(off, CH)], bins_[buf], sems[buf])

    def clear_tables():
        for i in range(16):
            histf[pl.ds(i * 16, 16)] = zeros
            cntf[pl.ds(i * 16, 16)] = zeros

    def fold(tab):
        acc = zeros
        for i in range(16):
            acc = acc + tab[pl.ds(i * 16, 16)]
        tab[pl.ds(0, 16)] = acc
        return acc + plsc.load_gather(tab, [(lanes + 8) % 16])

    clear_tables()
    start(0, 0)
    for t in range(total):
        buf = t % 2
        if t + 1 < total:
            start(t + 1, 1 - buf)
        pend[buf][0].wait()
        pend[buf][1].wait()
        mv, bv = mags[buf], bins_[buf]

        def body(i, carry, mv=mv, bv=bv):
            off = i * (16 * UNROLL)
            for u in range(UNROLL):
                mg = mv[pl.ds(off + u * 16, 16)]
                bn = bv[pl.ds(off + u * 16, 16)]
                flat = lane_bases[u] + bn
                plsc.addupdate_scatter(histf, [flat], mg)
                plsc.addupdate_scatter(cntf, [flat], ones)
            return carry

        lax.fori_loop(0, CH // (16 * UNROLL), body, 0)

        if (t + 1) % NCHUNK == 0:
            box = wid * per_w + (t // NCHUNK)
            htot = fold(histf)
            ctot = fold(cntf)
            nz = ctot != 0.0
            resv[...] = jnp.where(nz, htot / jnp.where(nz, ctot, 1.0), 0.0)
            pltpu.sync_copy(resv, out_hbm.at[box])
            if t + 1 < total:
                clear_tables()


def _sc_hist(mag, bins):
    M = mag.shape[0]
    return pl.kernel(
        _sc_body,
        mesh=plsc.VectorSubcoreMesh(core_axis_name="c", subcore_axis_name="s"),
        out_type=jax.ShapeDtypeStruct((M, 16), jnp.float32),
        scratch_types=[
            pltpu.VMEM((CH,), jnp.float32),
            pltpu.VMEM((CH,), jnp.float32),
            pltpu.VMEM((CH,), jnp.int32),
            pltpu.VMEM((CH,), jnp.int32),
            pltpu.VMEM((2 * 16 * N_BINS,), jnp.float32),
            pltpu.VMEM((2 * 16 * N_BINS,), jnp.float32),
            pltpu.VMEM((16,), jnp.float32),
            pltpu.SemaphoreType.DMA,
            pltpu.SemaphoreType.DMA,
        ],
        compiler_params=pltpu.CompilerParams(needs_layout_passes=False),
    )(mag, bins)


def kernel(flows, boxes):
    M = boxes.shape[0]
    mag, bins = pl.pallas_call(
        _tc_body,
        grid=(M,),
        in_specs=[
            pl.BlockSpec(memory_space=pltpu.SMEM),
            pl.BlockSpec((flows.shape[0], 2, H, W), lambda m: (0, 0, 0, 0)),
        ],
        out_specs=[
            pl.BlockSpec((1, OUT, OUT), lambda m: (m, 0, 0)),
            pl.BlockSpec((1, OUT, OUT), lambda m: (m, 0, 0)),
        ],
        out_shape=[
            jax.ShapeDtypeStruct((M, OUT, OUT), jnp.float32),
            jax.ShapeDtypeStruct((M, OUT, OUT), jnp.int32),
        ],
    )(boxes, flows.astype(jnp.bfloat16))
    out = _sc_hist(mag.reshape(M, P), bins.reshape(M, P))
    return out[:, :N_BINS]
